# baseline (speedup 1.0000x reference)
import jax
import jax.numpy as jnp
import numpy as np
from jax.experimental import pallas as pl
from jax.experimental.pallas import tpu as pltpu

_DIM = 128
_HEADS = 8
_HD = _DIM // _HEADS


def _fused_body(x_ref, wmu_ref, bmu_ref, inw_ref, inb_ref, ow_ref, ob_ref,
                wo_ref, bo_ref, out_ref):
    f32 = jnp.float32
    bf16 = jnp.bfloat16
    nb = x_ref.shape[0]
    S = x_ref.shape[1]
    w1 = wmu_ref[:]
    ones = jnp.ones((S, _HD), bf16)
    H = range(_HEADS)
    sl = [slice(h * _HD, (h + 1) * _HD) for h in H]
    lim = jnp.asarray(80.0, bf16)
    qkvs = []
    for i in range(nb):
        enh = jax.lax.dot_general(x_ref[i], w1, (((1,), (1,)), ((), ())),
                                  preferred_element_type=f32) + bmu_ref[:]
        qkvs.append((jax.lax.dot_general(enh, inw_ref[:],
                                         (((1,), (1,)), ((), ())),
                                         preferred_element_type=f32)
                     + inb_ref[:]).astype(bf16))
    ss = [jax.lax.dot_general(qkv[:, sl[h]], qkv[:, _DIM:][:, sl[h]],
                              (((1,), (1,)), ((), ())),
                              preferred_element_type=f32)
          for qkv in qkvs for h in H]
    es = [jnp.exp2(jnp.clip(s.astype(bf16), -lim, lim)) for s in ss]
    vaugs = [jnp.concatenate([qkv[:, 2 * _DIM:][:, sl[h]], ones], axis=1)
             for qkv in qkvs for h in H]
    pvs = [jax.lax.dot_general(e, va, (((1,), (0,)), ((), ())),
                               preferred_element_type=f32)
           for e, va in zip(es, vaugs)]
    outs = [(pv[:, :_HD] / pv[:, _HD:2 * _HD]).astype(bf16) for pv in pvs]
    for i in range(nb):
        o = jnp.concatenate(outs[i * _HEADS:(i + 1) * _HEADS], axis=1)
        att = jax.lax.dot_general(o, ow_ref[:], (((1,), (1,)), ((), ())),
                                  preferred_element_type=f32) + ob_ref[:]
        y = jax.lax.dot_general(att, wo_ref[:], (((1,), (1,)), ((), ())),
                                preferred_element_type=f32) + bo_ref[:]
        out_ref[i] = y


def _run(x, W_mu, b_mu, in_proj_w, in_proj_b, attn_out_w, attn_out_b, W_out,
         b_out):
    B, S, D = x.shape

    def full(shape):
        return pl.BlockSpec(shape, lambda b: (0,) * len(shape))

    nb = 2
    return pl.pallas_call(
        _fused_body,
        grid=(B // nb,),
        in_specs=[
            pl.BlockSpec((nb, S, D), lambda b: (b, 0, 0)),
            full(W_mu.shape),
            full((1, D)),
            full(in_proj_w.shape),
            full((1, 3 * D)),
            full(attn_out_w.shape),
            full((1, D)),
            full(W_out.shape),
            full((1, D)),
        ],
        out_specs=pl.BlockSpec((nb, S, D), lambda b: (b, 0, 0)),
        out_shape=jax.ShapeDtypeStruct((B, S, D), jnp.float32),
        compiler_params=pltpu.CompilerParams(
            dimension_semantics=("arbitrary",)),
    )(x, W_mu, b_mu, in_proj_w, in_proj_b, attn_out_w, attn_out_b, W_out,
      b_out)


def kernel(x, support_examples, support_labels, memory_bank, memory_usage,
           memory_labels, W_mu, b_mu, in_proj_w, in_proj_b, attn_out_w,
           attn_out_b, W_out, b_out):
    B, S, D = x.shape
    qscale = np.float32(np.log2(np.e) / np.sqrt(_HD))
    row_scale = jnp.concatenate(
        [jnp.full((D,), qscale, jnp.float32), jnp.ones((2 * D,), jnp.float32)])
    args = (W_mu[:, :D], b_mu.reshape(1, -1),
            in_proj_w * row_scale[:, None],
            (in_proj_b * row_scale).reshape(1, -1),
            attn_out_w.astype(jnp.bfloat16), attn_out_b.reshape(1, -1),
            W_out, b_out.reshape(1, -1))

    return _run(x, *args)


# speedup vs baseline: 1.1663x; 1.1663x over previous
import jax
import jax.numpy as jnp
import numpy as np
from jax.experimental import pallas as pl
from jax.experimental.pallas import tpu as pltpu

_DIM = 128
_HEADS = 8
_HD = _DIM // _HEADS


def _fused_body(x_ref, wmu_ref, bmu_ref, inw_ref, inb_ref, ow_ref, ob_ref,
                wo_ref, bo_ref, out_ref):
    f32 = jnp.float32
    bf16 = jnp.bfloat16
    nb = x_ref.shape[0]
    S = x_ref.shape[1]
    w1 = wmu_ref[:, :_DIM]
    ow_bf = ow_ref[:].astype(bf16)
    ones = jnp.ones((S, _HD), bf16)
    H = range(_HEADS)
    sl = [slice(h * _HD, (h + 1) * _HD) for h in H]
    lim = jnp.asarray(80.0, bf16)
    qscale = np.float32(np.log2(np.e) / np.sqrt(_HD))
    qs, kvs = [], []
    for i in range(nb):
        enh = jax.lax.dot_general(x_ref[i], w1, (((1,), (1,)), ((), ())),
                                  preferred_element_type=f32) + bmu_ref[:]
        qkv = jax.lax.dot_general(enh, inw_ref[:], (((1,), (1,)), ((), ())),
                                  preferred_element_type=f32) + inb_ref[:]
        qs.append((qkv[:, :_DIM] * qscale).astype(bf16))
        kvs.append(qkv[:, _DIM:].astype(bf16))
    ss = [jax.lax.dot_general(qs[i][:, sl[h]], kvs[i][:, sl[h]],
                              (((1,), (1,)), ((), ())),
                              preferred_element_type=f32)
          for i in range(nb) for h in H]
    es = [jnp.exp2(jnp.clip(s.astype(bf16), -lim, lim)) for s in ss]
    vaugs = [jnp.concatenate([kvs[i][:, _DIM:][:, sl[h]], ones], axis=1)
             for i in range(nb) for h in H]
    pvs = [jax.lax.dot_general(e, va, (((1,), (0,)), ((), ())),
                               preferred_element_type=f32)
           for e, va in zip(es, vaugs)]
    outs = [(pv[:, :_HD] / pv[:, _HD:2 * _HD]).astype(bf16) for pv in pvs]
    for i in range(nb):
        o = jnp.concatenate(outs[i * _HEADS:(i + 1) * _HEADS], axis=1)
        att = jax.lax.dot_general(o, ow_bf, (((1,), (1,)), ((), ())),
                                  preferred_element_type=f32) + ob_ref[:]
        y = jax.lax.dot_general(att, wo_ref[:], (((1,), (1,)), ((), ())),
                                preferred_element_type=f32) + bo_ref[:]
        out_ref[i] = y


def kernel(x, support_examples, support_labels, memory_bank, memory_usage,
           memory_labels, W_mu, b_mu, in_proj_w, in_proj_b, attn_out_w,
           attn_out_b, W_out, b_out):
    B, S, D = x.shape

    def full(shape):
        return pl.BlockSpec(shape, lambda b: (0,) * len(shape))

    nb = 2
    return pl.pallas_call(
        _fused_body,
        grid=(B // nb,),
        in_specs=[
            pl.BlockSpec((nb, S, D), lambda b: (b, 0, 0)),
            full(W_mu.shape),
            full((1, D)),
            full(in_proj_w.shape),
            full((1, 3 * D)),
            full(attn_out_w.shape),
            full((1, D)),
            full(W_out.shape),
            full((1, D)),
        ],
        out_specs=pl.BlockSpec((nb, S, D), lambda b: (b, 0, 0)),
        out_shape=jax.ShapeDtypeStruct((B, S, D), jnp.float32),
        compiler_params=pltpu.CompilerParams(
            dimension_semantics=("arbitrary",)),
    )(x, W_mu, b_mu.reshape(1, -1), in_proj_w, in_proj_b.reshape(1, -1),
      attn_out_w, attn_out_b.reshape(1, -1), W_out, b_out.reshape(1, -1))
